# baseline (speedup 1.0000x reference)
import functools

import jax
import jax.numpy as jnp
from jax import lax
from jax.experimental import pallas as pl
from jax.experimental.pallas import tpu as pltpu
from jax.experimental.pallas import tpu_sc as plsc

_NUM_CORES = 2
_NUM_SUBCORES = 16
_NUM_WORKERS = _NUM_CORES * _NUM_SUBCORES
_ROWS_PER_CHUNK = 4


@functools.partial(jax.jit, static_argnames=("n_chunks",))
def _sc_gather(idx, table, *, n_chunks):
    v, d = table.shape
    bsz = _NUM_WORKERS * n_chunks * _ROWS_PER_CHUNK
    seq = idx.shape[0] // bsz
    chunk = _ROWS_PER_CHUNK * seq
    b_per_w = chunk * n_chunks
    mesh = plsc.VectorSubcoreMesh(core_axis_name="c", subcore_axis_name="s")

    @functools.partial(
        pl.kernel,
        mesh=mesh,
        out_type=jax.ShapeDtypeStruct((bsz, seq, d), table.dtype),
        scratch_types=[
            pltpu.VMEM((b_per_w,), jnp.int32),
            pltpu.VMEM((_ROWS_PER_CHUNK, seq, d), table.dtype),
            pltpu.VMEM((_ROWS_PER_CHUNK, seq, d), table.dtype),
            pltpu.SemaphoreType.DMA,
            pltpu.SemaphoreType.DMA,
        ],
        compiler_params=pltpu.CompilerParams(use_tc_tiling_on_sc=False),
    )
    def run(idx_hbm, table_hbm, out_hbm, idx_v, rows0, rows1, sem0, sem1):
        wid = lax.axis_index("s") * _NUM_CORES + lax.axis_index("c")
        base = wid * b_per_w
        pltpu.sync_copy(idx_hbm.at[pl.ds(base, b_per_w)], idx_v)

        bufs = (rows0, rows1)
        sems = (sem0, sem1)

        def start_gathers(i, b):
            for j in range(_ROWS_PER_CHUNK):
                off = pl.multiple_of(i * chunk + j * seq, seq)
                pltpu.async_copy(
                    table_hbm.at[idx_v.at[pl.ds(off, seq)]],
                    bufs[b].at[j], sems[b])

        start_gathers(0, 0)

        def group(g, carry):
            for b in range(2):
                i = g * 2 + b

                @pl.when(i + 1 < n_chunks)
                def _():
                    start_gathers(i + 1, 1 - b)

                for j in range(_ROWS_PER_CHUNK):
                    off = pl.multiple_of(i * chunk + j * seq, seq)
                    pltpu.make_async_copy(
                        table_hbm.at[idx_v.at[pl.ds(off, seq)]],
                        bufs[b].at[j], sems[b]).wait()
                row0 = pl.multiple_of(
                    wid * n_chunks * _ROWS_PER_CHUNK + i * _ROWS_PER_CHUNK,
                    _ROWS_PER_CHUNK)
                pltpu.sync_copy(
                    bufs[b], out_hbm.at[pl.ds(row0, _ROWS_PER_CHUNK)])
            return carry

        lax.fori_loop(0, n_chunks // 2, group, 0)

    return run(idx, table)


def kernel(token_ids, embedding_matrix):
    bsz, seq = token_ids.shape
    b = bsz * seq
    v, d = embedding_matrix.shape
    idx = token_ids.reshape(b).astype(jnp.int32)
    table_rm = jax.lax.optimization_barrier(
        embedding_matrix.reshape(v // 2, 2 * d))
    table_lin = table_rm.reshape(v, d)
    n_chunks = bsz // (_NUM_WORKERS * _ROWS_PER_CHUNK)
    assert n_chunks * _NUM_WORKERS * _ROWS_PER_CHUNK == bsz
    assert n_chunks % 2 == 0
    return _sc_gather(idx, table_lin, n_chunks=n_chunks)


# speedup vs baseline: 1.0889x; 1.0889x over previous
import functools

import jax
import jax.numpy as jnp
from jax import lax
from jax.experimental import pallas as pl
from jax.experimental.pallas import tpu as pltpu
from jax.experimental.pallas import tpu_sc as plsc

_NUM_CORES = 2
_NUM_SUBCORES = 16
_NUM_WORKERS = _NUM_CORES * _NUM_SUBCORES


@functools.partial(jax.jit, static_argnames=("chunk", "n_chunks"))
def _sc_gather(idx, table, *, chunk, n_chunks):
    b_per_w = chunk * n_chunks
    d = table.shape[1]
    mesh = plsc.VectorSubcoreMesh(core_axis_name="c", subcore_axis_name="s")

    @functools.partial(
        pl.kernel,
        mesh=mesh,
        out_type=jax.ShapeDtypeStruct((b_per_w * _NUM_WORKERS, d), table.dtype),
        scratch_types=[
            pltpu.VMEM((b_per_w,), jnp.int32),
            pltpu.VMEM((chunk, d), table.dtype),
            pltpu.VMEM((chunk, d), table.dtype),
            pltpu.SemaphoreType.DMA,
            pltpu.SemaphoreType.DMA,
        ],
        compiler_params=pltpu.CompilerParams(use_tc_tiling_on_sc=False),
    )
    def run(idx_hbm, table_hbm, out_hbm, idx_v, rows0, rows1, sem0, sem1):
        wid = lax.axis_index("s") * _NUM_CORES + lax.axis_index("c")
        base = wid * b_per_w
        pltpu.sync_copy(idx_hbm.at[pl.ds(base, b_per_w)], idx_v)

        bufs = (rows0, rows1)
        sems = (sem0, sem1)

        def start_gather(i, b):
            off = pl.multiple_of(i * chunk, chunk)
            pltpu.async_copy(
                table_hbm.at[idx_v.at[pl.ds(off, chunk)]], bufs[b], sems[b])

        start_gather(0, 0)

        def group(g, carry):
            for b in range(2):
                i = g * 2 + b

                @pl.when(i + 1 < n_chunks)
                def _():
                    start_gather(i + 1, 1 - b)

                off = pl.multiple_of(i * chunk, chunk)
                pltpu.make_async_copy(
                    table_hbm.at[idx_v.at[pl.ds(off, chunk)]],
                    bufs[b], sems[b]).wait()
                pltpu.sync_copy(
                    bufs[b], out_hbm.at[pl.ds(base + i * chunk, chunk)])
            return carry

        lax.fori_loop(0, n_chunks // 2, group, 0)

    return run(idx, table)


def kernel(token_ids, embedding_matrix):
    bsz, seq = token_ids.shape
    b = bsz * seq
    v, d = embedding_matrix.shape
    idx_t = token_ids.T.reshape(b).astype(jnp.int32) * 2
    table128 = jnp.pad(embedding_matrix, ((0, 0), (0, 128 - d)))
    table_lin = table128.reshape(2 * v, d)
    chunk = 800
    n_chunks = b // (_NUM_WORKERS * chunk)
    assert n_chunks * chunk * _NUM_WORKERS == b and n_chunks % 2 == 0
    out2d = _sc_gather(idx_t, table_lin, chunk=chunk, n_chunks=n_chunks)
    return out2d.reshape(seq, bsz, d).transpose(1, 0, 2)
